# baseline (speedup 1.0000x reference)
import jax
import jax.numpy as jnp
from jax import lax
from jax.experimental import pallas as pl
from jax.experimental.pallas import tpu as pltpu
from jax.experimental.pallas import tpu_sc as plsc

F = 26
V1 = 1001
D = 16
B = 4096
LW = 128
RPL = 4
IPS = F * RPL

_INFO = plsc.get_sparse_core_info()
NC, NS = _INFO.num_cores, _INFO.num_subcores
NW = NC * NS
SPT = B // NW
CH = 2
NCHUNK = SPT // CH
NACC = 8

_PAIRS = tuple((i, j) for i in range(F) for j in range(i + 1, F))

_GDN = lax.GatherDimensionNumbers(
    offset_dims=(), collapsed_slice_dims=(0,), start_index_map=(0,))


def _shuffle16(v, idx):
    return lax.gather(v, idx[:, None], dimension_numbers=_GDN,
                      slice_sizes=(1,),
                      mode=lax.GatherScatterMode.PROMISE_IN_BOUNDS)


def _lane_sum(v):
    for dist in (8, 4, 2, 1):
        v = v + _shuffle16(v, lax.iota(jnp.int32, D) ^ dist)
    return v


def _tile_body(table, idx4, xpad, wext, out, idx_v, x_v, w_v, buf0, buf1,
               out_v, sem0, sem1):
    wid = lax.axis_index("s") * NC + lax.axis_index("c")
    pltpu.sync_copy(idx4.at[wid], idx_v)
    pltpu.sync_copy(xpad.at[wid], x_v)
    pltpu.sync_copy(wext, w_v)

    def start_chunk(c, buf, sem):
        for s in range(CH):
            pltpu.async_copy(
                table.at[idx_v.at[c * CH + s, pl.ds(0, IPS)]],
                buf.at[pl.ds(s * IPS, IPS)], sem)

    def wait_chunk(c, buf, sem):
        for s in range(CH):
            pltpu.make_async_copy(
                table.at[idx_v.at[c * CH + s, pl.ds(0, IPS)]],
                buf.at[pl.ds(s * IPS, IPS)], sem).wait()

    start_chunk(0, buf0, sem0)
    start_chunk(1, buf1, sem1)

    zero = jnp.zeros((D,), jnp.float32)
    iota = lax.iota(jnp.int32, D)

    def do_chunk(c, outvec, buf, sem):
        wait_chunk(c, buf, sem)

        def s_body(s, ov):
            rb = s * IPS
            accs = [zero] * NACC
            for k, (i, j) in enumerate(_PAIRS):
                a = buf[rb + RPL * i + j // 8, pl.ds(D * (j % 8), D)]
                bvec = buf[rb + RPL * j + i // 8, pl.ds(D * (i % 8), D)]
                accs[k % NACC] = accs[k % NACC] + a * bvec
            row = c * CH + s
            tot = (x_v[row, pl.ds(0, D)] * w_v[pl.ds(0, D)]
                   + x_v[row, pl.ds(D, D)] * w_v[pl.ds(D, D)])
            for a in accs:
                tot = tot + a
            r = _lane_sum(tot)
            return jnp.where(iota == row % D, r, ov)

        return lax.fori_loop(0, CH, s_body, outvec)

    def g_body(g, outvec):
        c0 = 2 * g
        c1 = 2 * g + 1
        outvec = do_chunk(c0, outvec, buf0, sem0)

        @pl.when(g < NCHUNK // 2 - 1)
        def _():
            start_chunk(c0 + 2, buf0, sem0)

        outvec = do_chunk(c1, outvec, buf1, sem1)

        @pl.when(g < NCHUNK // 2 - 1)
        def _():
            start_chunk(c1 + 2, buf1, sem1)

        flush = c1 % 8 == 7

        @pl.when(flush)
        def _():
            out_v[pl.ds((c1 - 7) * CH, D)] = outvec

        return jnp.where(flush, zero, outvec)

    lax.fori_loop(0, NCHUNK // 2, g_body, zero)
    pltpu.sync_copy(out_v, out.at[wid])


_sc_call = pl.kernel(
    _tile_body,
    out_type=jax.ShapeDtypeStruct((NW, SPT), jnp.float32),
    mesh=plsc.VectorSubcoreMesh(core_axis_name="c", subcore_axis_name="s"),
    scratch_types=[
        pltpu.VMEM((SPT, LW), jnp.int32),
        pltpu.VMEM((SPT, LW), jnp.float32),
        pltpu.VMEM((LW,), jnp.float32),
        pltpu.VMEM((CH * IPS, LW), jnp.float32),
        pltpu.VMEM((CH * IPS, LW), jnp.float32),
        pltpu.VMEM((SPT,), jnp.float32),
        pltpu.SemaphoreType.DMA,
        pltpu.SemaphoreType.DMA,
    ],
    compiler_params=pltpu.CompilerParams(use_tc_tiling_on_sc=False),
)


def kernel(sparse_inputs, emb, W, b):
    x = sparse_inputs.astype(jnp.int32)
    fzero = b[0] * 0.0
    table = emb.transpose(0, 2, 1, 3).reshape(F * V1, F * D)
    table = jnp.pad(table, ((0, 0), (0, RPL * LW - F * D)))
    table = (table + fzero).reshape(F * V1 * RPL, LW)
    base = (x + jnp.arange(F, dtype=jnp.int32) * V1) * RPL
    idx4 = base[:, :, None] + jnp.arange(RPL, dtype=jnp.int32)
    idx4 = jnp.pad(idx4.reshape(B, IPS), ((0, 0), (0, LW - IPS)))
    idx4 = (idx4.astype(jnp.float32) + fzero).astype(jnp.int32)
    idx4 = idx4.reshape(NW, SPT, LW)
    xf = x.astype(jnp.float32)
    xpad = jnp.concatenate(
        [xf, jnp.zeros((B, 5), jnp.float32), jnp.ones((B, 1), jnp.float32),
         jnp.zeros((B, LW - 32), jnp.float32)], axis=1) + fzero
    xpad = xpad.reshape(NW, SPT, LW)
    wext = jnp.concatenate(
        [W[:, 0], jnp.zeros((5,), jnp.float32), b,
         jnp.zeros((LW - 32,), jnp.float32)])
    out = _sc_call(table, idx4, xpad, wext)
    return out.reshape(B)


# speedup vs baseline: 1.1231x; 1.1231x over previous
import jax
import jax.numpy as jnp
from jax import lax
from jax.experimental import pallas as pl
from jax.experimental.pallas import tpu as pltpu
from jax.experimental.pallas import tpu_sc as plsc

F = 26
V1 = 1001
D = 16
B = 4096
RW = F * D
XS = 32
BIAS_SLOT = 1 + V1 * 31

_INFO = plsc.get_sparse_core_info()
NC, NS = _INFO.num_cores, _INFO.num_subcores
NW = NC * NS
SPT = B // NW
CH = 4
NCHUNK = SPT // CH
NACC = 8

_PAIRS = tuple((i, j) for i in range(F) for j in range(i + 1, F))

_GDN = lax.GatherDimensionNumbers(
    offset_dims=(), collapsed_slice_dims=(0,), start_index_map=(0,))


def _shuffle16(v, idx):
    return lax.gather(v, idx[:, None], dimension_numbers=_GDN,
                      slice_sizes=(1,),
                      mode=lax.GatherScatterMode.PROMISE_IN_BOUNDS)


def _lane_sum(v):
    for dist in (8, 4, 2, 1):
        v = v + _shuffle16(v, lax.iota(jnp.int32, D) ^ dist)
    return v


def _tile_body(table, idxs, wext, out, idx_v, w_v, buf0, buf1, out_v,
               sem0, sem1):
    wid = lax.axis_index("s") * NC + lax.axis_index("c")
    pltpu.sync_copy(idxs.at[pl.ds(wid * SPT * XS, SPT * XS)], idx_v)
    pltpu.sync_copy(wext, w_v)

    def start_chunk(c, buf, sem):
        for s in range(CH):
            pltpu.async_copy(
                table.at[idx_v.at[pl.ds((c * CH + s) * XS, F)]],
                buf.at[pl.ds(s * F, F)], sem)

    def wait_chunk(c, buf, sem):
        for s in range(CH):
            pltpu.make_async_copy(
                table.at[idx_v.at[pl.ds((c * CH + s) * XS, F)]],
                buf.at[pl.ds(s * F, F)], sem).wait()

    start_chunk(0, buf0, sem0)
    start_chunk(1, buf1, sem1)

    zero = jnp.zeros((D,), jnp.float32)
    iota = lax.iota(jnp.int32, D)

    def do_chunk(c, outvec, buf, sem):
        wait_chunk(c, buf, sem)

        def s_body(s, ov):
            rb = s * F
            accs = [zero] * NACC
            for k, (i, j) in enumerate(_PAIRS):
                a = buf[rb + i, pl.ds(D * j, D)]
                bvec = buf[rb + j, pl.ds(D * i, D)]
                accs[k % NACC] = accs[k % NACC] + a * bvec
            ib = (c * CH + s) * XS
            xf0 = (idx_v[pl.ds(ib, D)] - V1 * iota).astype(jnp.float32)
            xf1 = (idx_v[pl.ds(ib + D, D)] - V1 * (D + iota)).astype(
                jnp.float32)
            tot = xf0 * w_v[pl.ds(0, D)] + xf1 * w_v[pl.ds(D, D)]
            for a in accs:
                tot = tot + a
            r = _lane_sum(tot)
            lane = (c * CH + s) % D
            return jnp.where(iota == lane, r, ov)

        return lax.fori_loop(0, CH, s_body, outvec)

    def g_body(g, outvec):
        c0 = 2 * g
        c1 = 2 * g + 1
        outvec = do_chunk(c0, outvec, buf0, sem0)

        @pl.when(g < NCHUNK // 2 - 1)
        def _():
            start_chunk(c0 + 2, buf0, sem0)

        outvec = do_chunk(c1, outvec, buf1, sem1)

        @pl.when(g < NCHUNK // 2 - 1)
        def _():
            start_chunk(c1 + 2, buf1, sem1)

        flush = c1 % 4 == 3

        @pl.when(flush)
        def _():
            out_v[pl.ds((c1 - 3) * CH, D)] = outvec

        return jnp.where(flush, zero, outvec)

    lax.fori_loop(0, NCHUNK // 2, g_body, zero)
    pltpu.sync_copy(out_v, out.at[pl.ds(wid * SPT, SPT)])


_sc_call = pl.kernel(
    _tile_body,
    out_type=jax.ShapeDtypeStruct((B,), jnp.float32),
    mesh=plsc.VectorSubcoreMesh(core_axis_name="c", subcore_axis_name="s"),
    scratch_types=[
        pltpu.VMEM((SPT * XS,), jnp.int32),
        pltpu.VMEM((128,), jnp.float32),
        pltpu.VMEM((CH * F, RW), jnp.float32),
        pltpu.VMEM((CH * F, RW), jnp.float32),
        pltpu.VMEM((SPT,), jnp.float32),
        pltpu.SemaphoreType.DMA,
        pltpu.SemaphoreType.DMA,
    ],
    compiler_params=pltpu.CompilerParams(use_tc_tiling_on_sc=False),
)


def kernel(sparse_inputs, emb, W, b):
    x = sparse_inputs.astype(jnp.int32)
    table = emb.transpose(0, 2, 1, 3).reshape(F * V1, RW)
    idxp = x + jnp.arange(F, dtype=jnp.int32) * V1
    idxs = jnp.concatenate(
        [idxp, jnp.zeros((B, XS - F - 1), jnp.int32),
         jnp.full((B, 1), BIAS_SLOT, jnp.int32)], axis=1).reshape(B * XS)
    wext = jnp.concatenate(
        [W[:, 0], jnp.zeros((5,), jnp.float32), b,
         jnp.zeros((96,), jnp.float32)])
    return _sc_call(table, idxs, wext)
